# baseline (speedup 1.0000x reference)
import functools

import jax
import jax.numpy as jnp
from jax import lax
from jax.experimental import pallas as pl
from jax.experimental.pallas import tpu as pltpu
from jax.experimental.pallas import tpu_sc as plsc

N = 10000
E = 320000
ROWB = 1000
WIN = 128
NWORK = 32
WPT = 80
EPAD = NWORK * WPT * WIN
ZCH = 640
NJUNK = 16


def _dg(a, w):
    return lax.dot_general(
        a, w, (((1,), (1,)), ((), ())),
        preferred_element_type=jnp.float32)


@functools.lru_cache(maxsize=None)
def _make_segsum(d):
    mesh = plsc.VectorSubcoreMesh(core_axis_name="c", subcore_axis_name="s")
    assert d <= 64
    nbuf = 5 if d == 64 else 8
    ngrp = WPT // nbuf

    @functools.partial(
        pl.kernel,
        mesh=mesh,
        out_type=jax.ShapeDtypeStruct((2, N, d), jnp.float32),
        scratch_types=[
            pltpu.VMEM((WPT, WIN), jnp.int32),
            pltpu.VMEM((WPT, WIN), jnp.int32),
            pltpu.VMEM((nbuf, WIN, d), jnp.float32),
            pltpu.VMEM_SHARED((N + NJUNK, d), jnp.float32),
        ] + [pltpu.SemaphoreType.DMA] * (2 * nbuf),
        compiler_params=pltpu.CompilerParams(use_tc_tiling_on_sc=False),
    )
    def segsum(x_hbm, srcw_hbm, dstw_hbm, zr_hbm, out_hbm,
               sidx, didx, rows, acc_sh, *sems):
        sem_g = sems[:nbuf]
        sem_s = sems[nbuf:]
        c = lax.axis_index("c")
        s = lax.axis_index("s")
        wid = s * 2 + c

        pltpu.sync_copy(srcw_hbm.at[pl.ds(wid * WPT, WPT), :], sidx)
        pltpu.sync_copy(dstw_hbm.at[pl.ds(wid * WPT, WPT), :], didx)

        pltpu.sync_copy(zr_hbm, rows.at[0])

        def _zfire(r0, nrows):
            pltpu.async_copy(rows.at[0, pl.ds(0, nrows), :],
                             acc_sh.at[pl.ds(r0, nrows), :], sem_g[0])

        def _zdrain(r0, nrows):
            pltpu.make_async_copy(rows.at[0, pl.ds(0, nrows), :],
                                  acc_sh.at[pl.ds(r0, nrows), :],
                                  sem_g[0]).wait()

        @pl.when(s < 15)
        def _():
            for j in range(5):
                _zfire(s * ZCH + j * WIN, WIN)
            for j in range(5):
                _zdrain(s * ZCH + j * WIN, WIN)

        @pl.when(s == 15)
        def _():
            for j in range(3):
                _zfire(15 * ZCH + j * WIN, WIN)
            _zfire(15 * ZCH + 3 * WIN, 16)
            _zfire(N, NJUNK)
            for j in range(3):
                _zdrain(15 * ZCH + j * WIN, WIN)
            _zdrain(15 * ZCH + 3 * WIN, 16)
            _zdrain(N, NJUNK)

        plsc.subcore_barrier()

        def _gather(w, j):
            return pltpu.make_async_copy(x_hbm.at[sidx.at[w]], rows.at[j],
                                         sem_g[j])

        def _scatter(w, j):
            return pltpu.make_async_copy(rows.at[j], acc_sh.at[didx.at[w]],
                                         sem_s[j])

        def group(g, carry):
            wbase = g * nbuf
            for j in range(nbuf):
                @pl.when(g > 0)
                def _():
                    _scatter(wbase + j, j).wait()
                _gather(wbase + j, j).start()
            for j in range(nbuf):
                _gather(wbase + j, j).wait()
                _scatter(wbase + j, j).start(add=True)
            return carry

        lax.fori_loop(0, ngrp, group, 0)
        for j in range(nbuf):
            _scatter(j, j).wait()

        plsc.subcore_barrier()

        def _dfire(r0, nrows):
            pltpu.async_copy(acc_sh.at[pl.ds(r0, nrows), :],
                             out_hbm.at[c, pl.ds(r0, nrows), :], sem_s[0])

        def _ddrain(r0, nrows):
            pltpu.make_async_copy(acc_sh.at[pl.ds(r0, nrows), :],
                                  out_hbm.at[c, pl.ds(r0, nrows), :],
                                  sem_s[0]).wait()

        @pl.when(s < 15)
        def _():
            for j in range(5):
                _dfire(s * ZCH + j * WIN, WIN)
            for j in range(5):
                _ddrain(s * ZCH + j * WIN, WIN)

        @pl.when(s == 15)
        def _():
            for j in range(3):
                _dfire(15 * ZCH + j * WIN, WIN)
            _dfire(15 * ZCH + 3 * WIN, 16)
            for j in range(3):
                _ddrain(15 * ZCH + j * WIN, WIN)
            _ddrain(15 * ZCH + 3 * WIN, 16)

    return segsum


def _segsum(xin, srcw, dstw):
    d = xin.shape[1]
    zr = jnp.zeros((WIN, d), jnp.float32)
    return _make_segsum(d)(xin, srcw, dstw, zr)


@functools.lru_cache(maxsize=None)
def _make_segsum_colsplit(d):
    mesh = plsc.VectorSubcoreMesh(core_axis_name="c", subcore_axis_name="s")
    nbuf = 5 if d == 64 else 10
    wpt = 2 * WPT
    ngrp = wpt // nbuf

    @functools.partial(
        pl.kernel,
        mesh=mesh,
        out_type=jax.ShapeDtypeStruct((2, N, d), jnp.float32),
        scratch_types=[
            pltpu.VMEM((wpt, WIN), jnp.int32),
            pltpu.VMEM((wpt, WIN), jnp.int32),
            pltpu.VMEM((nbuf, WIN, d), jnp.float32),
            pltpu.VMEM_SHARED((N + NJUNK, d), jnp.float32),
        ] + [pltpu.SemaphoreType.DMA] * (2 * nbuf),
        compiler_params=pltpu.CompilerParams(use_tc_tiling_on_sc=False),
    )
    def segsum(xa_hbm, xb_hbm, srcw_hbm, dstw_hbm, zr_hbm, out_hbm,
               sidx, didx, rows, acc_sh, *sems):
        sem_g = sems[:nbuf]
        sem_s = sems[nbuf:]
        c = lax.axis_index("c")
        s = lax.axis_index("s")

        pltpu.sync_copy(srcw_hbm.at[pl.ds(s * wpt, wpt), :], sidx)
        pltpu.sync_copy(dstw_hbm.at[pl.ds(s * wpt, wpt), :], didx)

        pltpu.sync_copy(zr_hbm, rows.at[0])

        def _zfire(r0, nrows):
            pltpu.async_copy(rows.at[0, pl.ds(0, nrows), :],
                             acc_sh.at[pl.ds(r0, nrows), :], sem_g[0])

        def _zdrain(r0, nrows):
            pltpu.make_async_copy(rows.at[0, pl.ds(0, nrows), :],
                                  acc_sh.at[pl.ds(r0, nrows), :],
                                  sem_g[0]).wait()

        @pl.when(s < 15)
        def _():
            for j in range(5):
                _zfire(s * ZCH + j * WIN, WIN)
            for j in range(5):
                _zdrain(s * ZCH + j * WIN, WIN)

        @pl.when(s == 15)
        def _():
            for j in range(3):
                _zfire(15 * ZCH + j * WIN, WIN)
            _zfire(15 * ZCH + 3 * WIN, 16)
            _zfire(N, NJUNK)
            for j in range(3):
                _zdrain(15 * ZCH + j * WIN, WIN)
            _zdrain(15 * ZCH + 3 * WIN, 16)
            _zdrain(N, NJUNK)

        plsc.subcore_barrier()

        def _gather(x_hbm, w, j):
            return pltpu.make_async_copy(x_hbm.at[sidx.at[w]], rows.at[j],
                                         sem_g[j])

        def _scatter(w, j):
            return pltpu.make_async_copy(rows.at[j], acc_sh.at[didx.at[w]],
                                         sem_s[j])

        def _loop(x_hbm):
            def group(g, carry):
                wbase = g * nbuf
                for j in range(nbuf):
                    @pl.when(g > 0)
                    def _():
                        _scatter(wbase + j, j).wait()
                    _gather(x_hbm, wbase + j, j).start()
                for j in range(nbuf):
                    _gather(x_hbm, wbase + j, j).wait()
                    _scatter(wbase + j, j).start(add=True)
                return carry
            lax.fori_loop(0, ngrp, group, 0)

        @pl.when(c == 0)
        def _():
            _loop(xa_hbm)

        @pl.when(c == 1)
        def _():
            _loop(xb_hbm)

        for j in range(nbuf):
            _scatter(j, j).wait()

        plsc.subcore_barrier()

        def _dfire(r0, nrows):
            pltpu.async_copy(acc_sh.at[pl.ds(r0, nrows), :],
                             out_hbm.at[c, pl.ds(r0, nrows), :], sem_s[0])

        def _ddrain(r0, nrows):
            pltpu.make_async_copy(acc_sh.at[pl.ds(r0, nrows), :],
                                  out_hbm.at[c, pl.ds(r0, nrows), :],
                                  sem_s[0]).wait()

        @pl.when(s < 15)
        def _():
            for j in range(5):
                _dfire(s * ZCH + j * WIN, WIN)
            for j in range(5):
                _ddrain(s * ZCH + j * WIN, WIN)

        @pl.when(s == 15)
        def _():
            for j in range(3):
                _dfire(15 * ZCH + j * WIN, WIN)
            _dfire(15 * ZCH + 3 * WIN, 16)
            for j in range(3):
                _ddrain(15 * ZCH + j * WIN, WIN)
            _ddrain(15 * ZCH + 3 * WIN, 16)

    return segsum


def _colsplit(xa, xb, srcw, dstw):
    d = xa.shape[1]
    zr = jnp.zeros((WIN, d), jnp.float32)
    return _make_segsum_colsplit(d)(xa, xb, srcw, dstw, zr)


def _row_spec(d):
    return pl.BlockSpec((ROWB, d), lambda i: (i, 0))


def _seg_spec(d):
    return pl.BlockSpec((2, ROWB, d), lambda i: (0, i, 0))


_CNT_SPEC = pl.BlockSpec((2, 1, 1, ROWB), lambda i: (0, i, 0, 0))


def _full_spec(shape):
    rank = len(shape)
    return pl.BlockSpec(shape, lambda i: (0,) * rank)


def _tc_call(body, in_arrays, in_specs, out_d):
    if isinstance(out_d, (tuple, list)):
        out_shape = tuple(jax.ShapeDtypeStruct((N, dd), jnp.float32)
                          for dd in out_d)
        out_specs = tuple(_row_spec(dd) for dd in out_d)
    else:
        out_shape = jax.ShapeDtypeStruct((N, out_d), jnp.float32)
        out_specs = _row_spec(out_d)
    return pl.pallas_call(
        body,
        grid=(N // ROWB,),
        in_specs=in_specs,
        out_specs=out_specs,
        out_shape=out_shape,
    )(*in_arrays)


def _invc(cnt_ref):
    cnt = cnt_ref[0, 0, 0, :] + cnt_ref[1, 0, 0, :]
    return 1.0 / jnp.maximum(cnt, 1.0)


def _mean(seg_ref, cnt_ref):
    ssum = seg_ref[0] + seg_ref[1]
    return ssum * _invc(cnt_ref)[:, None]


def _stage_a(x_r, w_r, b_r, o_r):
    x0 = jnp.tanh(_dg(x_r[...], w_r[...]) + b_r[...])
    o_r[...] = jnp.concatenate(
        [x0, jnp.ones((ROWB, 16), jnp.float32)], axis=1)


def _stage_p1(x0_r, wr_r, bl_r, p_r):
    p_r[...] = _dg(x0_r[:, :32], wr_r[...]) + bl_r[...]


def _stage_b(s_r, c_r, p_r, wl_r, o_r):
    ssum = s_r[0, :, :32] + s_r[1, :, :32]
    mean = ssum * _invc(c_r)[:, None]
    o_r[...] = jnp.tanh(_dg(mean, wl_r[...]) + p_r[...])


def _stage_p2(x1_r, wr_r, bl_r, p_r):
    p_r[...] = _dg(x1_r[...], wr_r[...]) + bl_r[...]


def _stage_c(s_r, c_r, p_r, wl_r, oa_r, ob_r):
    mean = _mean(s_r, c_r)
    x2 = jnp.tanh(_dg(mean, wl_r[...]) + p_r[...])
    oa_r[...] = x2[:, :64]
    ob_r[...] = x2[:, 64:]


def _stage_p3(x2a_r, x2b_r, x1_r, bnwr_r, bnbl_r, d1wl_r, s1w_r, s1b_r,
              p_r, g3p_r, sk1_r):
    p_r[...] = (_dg(x2a_r[...], bnwr_r[:, :64]) +
                _dg(x2b_r[...], bnwr_r[:, 64:]) + bnbl_r[...])
    g3p_r[...] = (_dg(x2a_r[...], d1wl_r[:, 128:192]) +
                  _dg(x2b_r[...], d1wl_r[:, 192:]))
    sk1_r[...] = _dg(x1_r[...], s1w_r[...]) + s1b_r[...]


def _stage_d(s_r, c_r, p_r, g3p_r, bnwl_r, d1wl_r, x3_r, g3_r):
    invc = _invc(c_r)[:, None]
    x3 = jnp.tanh(_dg(s_r[0] * invc, bnwl_r[:, :64]) +
                  _dg(s_r[1] * invc, bnwl_r[:, 64:]) + p_r[...])
    x3_r[...] = x3
    g3_r[...] = _dg(x3, d1wl_r[:, :128]) + g3p_r[...]


def _stage_p4(x3_r, x2a_r, x2b_r, sk1_r, d1wr_r, d1bl_r, d2wl_r,
              p_r, g4p_r):
    p_r[...] = (_dg(x3_r[...], d1wr_r[:, :128]) +
                _dg(x2a_r[...], d1wr_r[:, 128:192]) +
                _dg(x2b_r[...], d1wr_r[:, 192:]) + d1bl_r[...])
    g4p_r[...] = _dg(sk1_r[...], d2wl_r[:, 64:])


def _stage_e(s_r, c_r, p_r, g4p_r, d2wl_r, h4_r, g4_r):
    h4 = jnp.tanh(_mean(s_r, c_r) + p_r[...])
    h4_r[...] = h4
    g4_r[...] = _dg(h4, d2wl_r[:, :64]) + g4p_r[...]


def _stage_p5(h4_r, sk1_r, x0_r, d2wr_r, d2bl_r, s2w_r, s2b_r,
              p_r, sk2_r):
    p_r[...] = (_dg(h4_r[...], d2wr_r[:, :64]) +
                _dg(sk1_r[...], d2wr_r[:, 64:]) + d2bl_r[...])
    sk2_r[...] = _dg(x0_r[:, :32], s2w_r[...]) + s2b_r[...]


def _stage_f(s_r, c_r, p_r, sk2_r, outw_r, outb_r, o_r):
    h5 = jnp.tanh(_mean(s_r, c_r) + p_r[...])
    o_r[...] = (_dg(h5, outw_r[:, :32]) + _dg(sk2_r[...], outw_r[:, 32:]) +
                outb_r[...])


def kernel(x, edge_index, fc_W, fc_b, c1_Wl, c1_bl, c1_Wr, c2_Wl, c2_bl,
           c2_Wr, bn_Wl, bn_bl, bn_Wr, d1_Wl, d1_bl, d1_Wr, d2_Wl, d2_bl,
           d2_Wr, out_W, out_b, s1_W, s1_b, s2_W, s2_b):
    pad = EPAD - E
    pada = jnp.arange(pad, dtype=jnp.int32)
    srcw = jnp.concatenate([edge_index[0], pada % 256]).reshape(-1, WIN)
    dstw = jnp.concatenate([edge_index[1], N + (pada % NJUNK)]).reshape(-1, WIN)

    r = lambda b: b.reshape(1, -1)

    x0a = _tc_call(
        _stage_a, (x, fc_W, r(fc_b)),
        [_row_spec(128), _full_spec((32, 128)), _full_spec((1, 32))], 48)

    s0 = _segsum(x0a, srcw, dstw)
    p1 = _tc_call(
        _stage_p1, (x0a, c1_Wr, r(c1_bl)),
        [_row_spec(48), _full_spec((64, 32)), _full_spec((1, 64))], 64)
    cnt = s0[:, :, 32].reshape(2, N // ROWB, 1, ROWB)
    x1 = _tc_call(
        _stage_b, (s0, cnt, p1, c1_Wl),
        [_seg_spec(48), _CNT_SPEC, _row_spec(64), _full_spec((64, 32))], 64)

    s1 = _segsum(x1, srcw, dstw)
    p2 = _tc_call(
        _stage_p2, (x1, c2_Wr, r(c2_bl)),
        [_row_spec(64), _full_spec((128, 64)), _full_spec((1, 128))], 128)
    x2a, x2b = _tc_call(
        _stage_c, (s1, cnt, p2, c2_Wl),
        [_seg_spec(64), _CNT_SPEC, _row_spec(128), _full_spec((128, 64))],
        (64, 64))

    s2m = _colsplit(x2a, x2b, srcw, dstw)
    p3, g3p, skip1 = _tc_call(
        _stage_p3, (x2a, x2b, x1, bn_Wr, r(bn_bl), d1_Wl, s1_W, r(s1_b)),
        [_row_spec(64), _row_spec(64), _row_spec(64),
         _full_spec((128, 128)), _full_spec((1, 128)),
         _full_spec((64, 256)), _full_spec((64, 64)), _full_spec((1, 64))],
        (128, 64, 64))
    x3, g3 = _tc_call(
        _stage_d, (s2m, cnt, p3, g3p, bn_Wl, d1_Wl),
        [_seg_spec(64), _CNT_SPEC, _row_spec(128), _row_spec(64),
         _full_spec((128, 128)), _full_spec((64, 256))], (128, 64))

    s3 = _segsum(g3, srcw, dstw)
    p4, g4p = _tc_call(
        _stage_p4, (x3, x2a, x2b, skip1, d1_Wr, r(d1_bl), d2_Wl),
        [_row_spec(128), _row_spec(64), _row_spec(64), _row_spec(64),
         _full_spec((64, 256)), _full_spec((1, 64)),
         _full_spec((32, 128))], (64, 32))
    h4, g4 = _tc_call(
        _stage_e, (s3, cnt, p4, g4p, d2_Wl),
        [_seg_spec(64), _CNT_SPEC, _row_spec(64), _row_spec(32),
         _full_spec((32, 128))], (64, 32))

    s4 = _segsum(g4, srcw, dstw)
    p5, skip2 = _tc_call(
        _stage_p5, (h4, skip1, x0a, d2_Wr, r(d2_bl), s2_W, r(s2_b)),
        [_row_spec(64), _row_spec(64), _row_spec(48),
         _full_spec((32, 128)), _full_spec((1, 32)),
         _full_spec((32, 32)), _full_spec((1, 32))], (32, 32))
    o = _tc_call(
        _stage_f, (s4, cnt, p5, skip2, out_W, r(out_b)),
        [_seg_spec(32), _CNT_SPEC, _row_spec(32), _row_spec(32),
         _full_spec((3, 64)), _full_spec((1, 3))], 3)

    return o


# speedup vs baseline: 1.0106x; 1.0106x over previous
import functools

import jax
import jax.numpy as jnp
from jax import lax
from jax.experimental import pallas as pl
from jax.experimental.pallas import tpu as pltpu
from jax.experimental.pallas import tpu_sc as plsc

N = 10000
E = 320000
ROWB = 1000
WIN = 128
NWORK = 32
WPT = 80
EPAD = NWORK * WPT * WIN
ZCH = 640
NJUNK = 16


def _dg(a, w):
    return lax.dot_general(
        a, w, (((1,), (1,)), ((), ())),
        preferred_element_type=jnp.float32)


@functools.lru_cache(maxsize=None)
def _make_segsum(d):
    mesh = plsc.VectorSubcoreMesh(core_axis_name="c", subcore_axis_name="s")
    assert d <= 64
    nbuf = 5 if d == 64 else 8
    ngrp = WPT // nbuf

    @functools.partial(
        pl.kernel,
        mesh=mesh,
        out_type=jax.ShapeDtypeStruct((2, N, d), jnp.float32),
        scratch_types=[
            pltpu.VMEM((WPT, WIN), jnp.int32),
            pltpu.VMEM((WPT, WIN), jnp.int32),
            pltpu.VMEM((nbuf, WIN, d), jnp.float32),
            pltpu.VMEM_SHARED((N + NJUNK, d), jnp.float32),
        ] + [pltpu.SemaphoreType.DMA] * (2 * nbuf),
        compiler_params=pltpu.CompilerParams(use_tc_tiling_on_sc=False),
    )
    def segsum(x_hbm, srcw_hbm, dstw_hbm, zr_hbm, out_hbm,
               sidx, didx, rows, acc_sh, *sems):
        sem_g = sems[:nbuf]
        sem_s = sems[nbuf:]
        c = lax.axis_index("c")
        s = lax.axis_index("s")
        wid = s * 2 + c

        pltpu.sync_copy(srcw_hbm.at[pl.ds(wid * WPT, WPT), :], sidx)
        pltpu.sync_copy(dstw_hbm.at[pl.ds(wid * WPT, WPT), :], didx)

        pltpu.sync_copy(zr_hbm, rows.at[0])

        def _zfire(r0, nrows):
            pltpu.async_copy(rows.at[0, pl.ds(0, nrows), :],
                             acc_sh.at[pl.ds(r0, nrows), :], sem_g[0])

        def _zdrain(r0, nrows):
            pltpu.make_async_copy(rows.at[0, pl.ds(0, nrows), :],
                                  acc_sh.at[pl.ds(r0, nrows), :],
                                  sem_g[0]).wait()

        @pl.when(s < 15)
        def _():
            for j in range(5):
                _zfire(s * ZCH + j * WIN, WIN)
            for j in range(5):
                _zdrain(s * ZCH + j * WIN, WIN)

        @pl.when(s == 15)
        def _():
            for j in range(3):
                _zfire(15 * ZCH + j * WIN, WIN)
            _zfire(15 * ZCH + 3 * WIN, 16)
            _zfire(N, NJUNK)
            for j in range(3):
                _zdrain(15 * ZCH + j * WIN, WIN)
            _zdrain(15 * ZCH + 3 * WIN, 16)
            _zdrain(N, NJUNK)

        plsc.subcore_barrier()

        def _gather(w, j):
            return pltpu.make_async_copy(x_hbm.at[sidx.at[w]], rows.at[j],
                                         sem_g[j])

        def _scatter(w, j):
            return pltpu.make_async_copy(rows.at[j], acc_sh.at[didx.at[w]],
                                         sem_s[j])

        def group(g, carry):
            wbase = g * nbuf
            for j in range(nbuf):
                @pl.when(g > 0)
                def _():
                    _scatter(wbase + j, j).wait()
                _gather(wbase + j, j).start()
            for j in range(nbuf):
                _gather(wbase + j, j).wait()
                _scatter(wbase + j, j).start(add=True)
            return carry

        lax.fori_loop(0, ngrp, group, 0)
        for j in range(nbuf):
            _scatter(j, j).wait()

        plsc.subcore_barrier()

        def _dfire(r0, nrows):
            pltpu.async_copy(acc_sh.at[pl.ds(r0, nrows), :],
                             out_hbm.at[c, pl.ds(r0, nrows), :], sem_s[0])

        def _ddrain(r0, nrows):
            pltpu.make_async_copy(acc_sh.at[pl.ds(r0, nrows), :],
                                  out_hbm.at[c, pl.ds(r0, nrows), :],
                                  sem_s[0]).wait()

        @pl.when(s < 15)
        def _():
            for j in range(5):
                _dfire(s * ZCH + j * WIN, WIN)
            for j in range(5):
                _ddrain(s * ZCH + j * WIN, WIN)

        @pl.when(s == 15)
        def _():
            for j in range(3):
                _dfire(15 * ZCH + j * WIN, WIN)
            _dfire(15 * ZCH + 3 * WIN, 16)
            for j in range(3):
                _ddrain(15 * ZCH + j * WIN, WIN)
            _ddrain(15 * ZCH + 3 * WIN, 16)

    return segsum


def _segsum(xin, srcw, dstw):
    d = xin.shape[1]
    zr = jnp.zeros((WIN, d), jnp.float32)
    return _make_segsum(d)(xin, srcw, dstw, zr)


@functools.lru_cache(maxsize=None)
def _make_segsum_colsplit(d):
    mesh = plsc.VectorSubcoreMesh(core_axis_name="c", subcore_axis_name="s")
    nbuf = 5 if d == 64 else 10
    wpt = 2 * WPT
    ngrp = wpt // nbuf

    @functools.partial(
        pl.kernel,
        mesh=mesh,
        out_type=jax.ShapeDtypeStruct((2, N, d), jnp.float32),
        scratch_types=[
            pltpu.VMEM((wpt, WIN), jnp.int32),
            pltpu.VMEM((wpt, WIN), jnp.int32),
            pltpu.VMEM((nbuf, WIN, d), jnp.float32),
            pltpu.VMEM_SHARED((N + NJUNK, d), jnp.float32),
        ] + [pltpu.SemaphoreType.DMA] * (2 * nbuf),
        compiler_params=pltpu.CompilerParams(use_tc_tiling_on_sc=False),
    )
    def segsum(xa_hbm, xb_hbm, srcw_hbm, dstw_hbm, zr_hbm, out_hbm,
               sidx, didx, rows, acc_sh, *sems):
        sem_g = sems[:nbuf]
        sem_s = sems[nbuf:]
        c = lax.axis_index("c")
        s = lax.axis_index("s")

        pltpu.sync_copy(srcw_hbm.at[pl.ds(s * wpt, wpt), :], sidx)
        pltpu.sync_copy(dstw_hbm.at[pl.ds(s * wpt, wpt), :], didx)

        pltpu.sync_copy(zr_hbm, rows.at[0])

        def _zfire(r0, nrows):
            pltpu.async_copy(rows.at[0, pl.ds(0, nrows), :],
                             acc_sh.at[pl.ds(r0, nrows), :], sem_g[0])

        def _zdrain(r0, nrows):
            pltpu.make_async_copy(rows.at[0, pl.ds(0, nrows), :],
                                  acc_sh.at[pl.ds(r0, nrows), :],
                                  sem_g[0]).wait()

        @pl.when(s < 15)
        def _():
            for j in range(5):
                _zfire(s * ZCH + j * WIN, WIN)
            for j in range(5):
                _zdrain(s * ZCH + j * WIN, WIN)

        @pl.when(s == 15)
        def _():
            for j in range(3):
                _zfire(15 * ZCH + j * WIN, WIN)
            _zfire(15 * ZCH + 3 * WIN, 16)
            _zfire(N, NJUNK)
            for j in range(3):
                _zdrain(15 * ZCH + j * WIN, WIN)
            _zdrain(15 * ZCH + 3 * WIN, 16)
            _zdrain(N, NJUNK)

        plsc.subcore_barrier()

        def _gather(x_hbm, w, j):
            return pltpu.make_async_copy(x_hbm.at[sidx.at[w]], rows.at[j],
                                         sem_g[j])

        def _scatter(w, j):
            return pltpu.make_async_copy(rows.at[j], acc_sh.at[didx.at[w]],
                                         sem_s[j])

        def _loop(x_hbm):
            def group(g, carry):
                wbase = g * nbuf
                for j in range(nbuf):
                    @pl.when(g > 0)
                    def _():
                        _scatter(wbase + j, j).wait()
                    _gather(x_hbm, wbase + j, j).start()
                for j in range(nbuf):
                    _gather(x_hbm, wbase + j, j).wait()
                    _scatter(wbase + j, j).start(add=True)
                return carry
            lax.fori_loop(0, ngrp, group, 0)

        @pl.when(c == 0)
        def _():
            _loop(xa_hbm)

        @pl.when(c == 1)
        def _():
            _loop(xb_hbm)

        for j in range(nbuf):
            _scatter(j, j).wait()

        plsc.subcore_barrier()

        def _dfire(r0, nrows):
            pltpu.async_copy(acc_sh.at[pl.ds(r0, nrows), :],
                             out_hbm.at[c, pl.ds(r0, nrows), :], sem_s[0])

        def _ddrain(r0, nrows):
            pltpu.make_async_copy(acc_sh.at[pl.ds(r0, nrows), :],
                                  out_hbm.at[c, pl.ds(r0, nrows), :],
                                  sem_s[0]).wait()

        @pl.when(s < 15)
        def _():
            for j in range(5):
                _dfire(s * ZCH + j * WIN, WIN)
            for j in range(5):
                _ddrain(s * ZCH + j * WIN, WIN)

        @pl.when(s == 15)
        def _():
            for j in range(3):
                _dfire(15 * ZCH + j * WIN, WIN)
            _dfire(15 * ZCH + 3 * WIN, 16)
            for j in range(3):
                _ddrain(15 * ZCH + j * WIN, WIN)
            _ddrain(15 * ZCH + 3 * WIN, 16)

    return segsum


def _colsplit(xa, xb, srcw, dstw):
    d = xa.shape[1]
    zr = jnp.zeros((WIN, d), jnp.float32)
    return _make_segsum_colsplit(d)(xa, xb, srcw, dstw, zr)


def _row_spec(d):
    return pl.BlockSpec((ROWB, d), lambda i: (i, 0))


def _seg_spec(d):
    return pl.BlockSpec((2, ROWB, d), lambda i: (0, i, 0))


_INV_SPEC = pl.BlockSpec((ROWB, 8), lambda i: (i, 0))


def _full_spec(shape):
    rank = len(shape)
    return pl.BlockSpec(shape, lambda i: (0,) * rank)


def _tc_call(body, in_arrays, in_specs, out_d):
    if isinstance(out_d, (tuple, list)):
        out_shape = tuple(jax.ShapeDtypeStruct((N, dd), jnp.float32)
                          for dd in out_d)
        out_specs = tuple(_row_spec(dd) for dd in out_d)
    else:
        out_shape = jax.ShapeDtypeStruct((N, out_d), jnp.float32)
        out_specs = _row_spec(out_d)
    return pl.pallas_call(
        body,
        grid=(N // ROWB,),
        in_specs=in_specs,
        out_specs=out_specs,
        out_shape=out_shape,
    )(*in_arrays)


def _mean(seg_ref, inv_r):
    ssum = seg_ref[0] + seg_ref[1]
    return ssum * inv_r[:, :1]


def _stage_a(x_r, w_r, b_r, o_r):
    x0 = jnp.tanh(_dg(x_r[...], w_r[...]) + b_r[...])
    o_r[...] = jnp.concatenate(
        [x0, jnp.ones((ROWB, 16), jnp.float32)], axis=1)


def _stage_p1(x0_r, wr_r, bl_r, p_r):
    p_r[...] = _dg(x0_r[:, :32], wr_r[...]) + bl_r[...]


def _stage_b(s_r, p_r, wl_r, o_r, iv_r):
    cnt = s_r[0, :, 32] + s_r[1, :, 32]
    invc = 1.0 / jnp.maximum(cnt, 1.0)
    ssum = s_r[0, :, :32] + s_r[1, :, :32]
    mean = ssum * invc[:, None]
    o_r[...] = jnp.tanh(_dg(mean, wl_r[...]) + p_r[...])
    iv_r[...] = jnp.broadcast_to(invc[:, None], (ROWB, 8))


def _stage_p2(x1_r, wr_r, bl_r, p_r):
    p_r[...] = _dg(x1_r[...], wr_r[...]) + bl_r[...]


def _stage_c(s_r, c_r, p_r, wl_r, oa_r, ob_r):
    mean = _mean(s_r, c_r)
    x2 = jnp.tanh(_dg(mean, wl_r[...]) + p_r[...])
    oa_r[...] = x2[:, :64]
    ob_r[...] = x2[:, 64:]


def _stage_p3(x2a_r, x2b_r, x1_r, bnwr_r, bnbl_r, d1wl_r, s1w_r, s1b_r,
              p_r, g3p_r, sk1_r):
    p_r[...] = (_dg(x2a_r[...], bnwr_r[:, :64]) +
                _dg(x2b_r[...], bnwr_r[:, 64:]) + bnbl_r[...])
    g3p_r[...] = (_dg(x2a_r[...], d1wl_r[:, 128:192]) +
                  _dg(x2b_r[...], d1wl_r[:, 192:]))
    sk1_r[...] = _dg(x1_r[...], s1w_r[...]) + s1b_r[...]


def _stage_d(s_r, c_r, p_r, g3p_r, bnwl_r, d1wl_r, x3_r, g3_r):
    invc = c_r[:, :1]
    x3 = jnp.tanh(_dg(s_r[0] * invc, bnwl_r[:, :64]) +
                  _dg(s_r[1] * invc, bnwl_r[:, 64:]) + p_r[...])
    x3_r[...] = x3
    g3_r[...] = _dg(x3, d1wl_r[:, :128]) + g3p_r[...]


def _stage_p4(x3_r, x2a_r, x2b_r, sk1_r, d1wr_r, d1bl_r, d2wl_r,
              p_r, g4p_r):
    p_r[...] = (_dg(x3_r[...], d1wr_r[:, :128]) +
                _dg(x2a_r[...], d1wr_r[:, 128:192]) +
                _dg(x2b_r[...], d1wr_r[:, 192:]) + d1bl_r[...])
    g4p_r[...] = _dg(sk1_r[...], d2wl_r[:, 64:])


def _stage_e(s_r, c_r, p_r, g4p_r, d2wl_r, h4_r, g4_r):
    h4 = jnp.tanh(_mean(s_r, c_r) + p_r[...])
    h4_r[...] = h4
    g4_r[...] = _dg(h4, d2wl_r[:, :64]) + g4p_r[...]


def _stage_p5(h4_r, sk1_r, x0_r, d2wr_r, d2bl_r, s2w_r, s2b_r,
              p_r, sk2_r):
    p_r[...] = (_dg(h4_r[...], d2wr_r[:, :64]) +
                _dg(sk1_r[...], d2wr_r[:, 64:]) + d2bl_r[...])
    sk2_r[...] = _dg(x0_r[:, :32], s2w_r[...]) + s2b_r[...]


def _stage_f(s_r, c_r, p_r, sk2_r, outw_r, outb_r, o_r):
    h5 = jnp.tanh(_mean(s_r, c_r) + p_r[...])
    o_r[...] = (_dg(h5, outw_r[:, :32]) + _dg(sk2_r[...], outw_r[:, 32:]) +
                outb_r[...])


def kernel(x, edge_index, fc_W, fc_b, c1_Wl, c1_bl, c1_Wr, c2_Wl, c2_bl,
           c2_Wr, bn_Wl, bn_bl, bn_Wr, d1_Wl, d1_bl, d1_Wr, d2_Wl, d2_bl,
           d2_Wr, out_W, out_b, s1_W, s1_b, s2_W, s2_b):
    pad = EPAD - E
    pada = jnp.arange(pad, dtype=jnp.int32)
    srcw = jnp.concatenate([edge_index[0], pada % 256]).reshape(-1, WIN)
    dstw = jnp.concatenate([edge_index[1], N + (pada % NJUNK)]).reshape(-1, WIN)

    r = lambda b: b.reshape(1, -1)

    x0a = _tc_call(
        _stage_a, (x, fc_W, r(fc_b)),
        [_row_spec(128), _full_spec((32, 128)), _full_spec((1, 32))], 48)

    s0 = _segsum(x0a, srcw, dstw)
    p1 = _tc_call(
        _stage_p1, (x0a, c1_Wr, r(c1_bl)),
        [_row_spec(48), _full_spec((64, 32)), _full_spec((1, 64))], 64)
    x1, inv = _tc_call(
        _stage_b, (s0, p1, c1_Wl),
        [_seg_spec(48), _row_spec(64), _full_spec((64, 32))], (64, 8))

    s1 = _segsum(x1, srcw, dstw)
    p2 = _tc_call(
        _stage_p2, (x1, c2_Wr, r(c2_bl)),
        [_row_spec(64), _full_spec((128, 64)), _full_spec((1, 128))], 128)
    x2a, x2b = _tc_call(
        _stage_c, (s1, inv, p2, c2_Wl),
        [_seg_spec(64), _INV_SPEC, _row_spec(128), _full_spec((128, 64))],
        (64, 64))

    s2m = _colsplit(x2a, x2b, srcw, dstw)
    p3, g3p, skip1 = _tc_call(
        _stage_p3, (x2a, x2b, x1, bn_Wr, r(bn_bl), d1_Wl, s1_W, r(s1_b)),
        [_row_spec(64), _row_spec(64), _row_spec(64),
         _full_spec((128, 128)), _full_spec((1, 128)),
         _full_spec((64, 256)), _full_spec((64, 64)), _full_spec((1, 64))],
        (128, 64, 64))
    x3, g3 = _tc_call(
        _stage_d, (s2m, inv, p3, g3p, bn_Wl, d1_Wl),
        [_seg_spec(64), _INV_SPEC, _row_spec(128), _row_spec(64),
         _full_spec((128, 128)), _full_spec((64, 256))], (128, 64))

    s3 = _segsum(g3, srcw, dstw)
    p4, g4p = _tc_call(
        _stage_p4, (x3, x2a, x2b, skip1, d1_Wr, r(d1_bl), d2_Wl),
        [_row_spec(128), _row_spec(64), _row_spec(64), _row_spec(64),
         _full_spec((64, 256)), _full_spec((1, 64)),
         _full_spec((32, 128))], (64, 32))
    h4, g4 = _tc_call(
        _stage_e, (s3, inv, p4, g4p, d2_Wl),
        [_seg_spec(64), _INV_SPEC, _row_spec(64), _row_spec(32),
         _full_spec((32, 128))], (64, 32))

    s4 = _segsum(g4, srcw, dstw)
    p5, skip2 = _tc_call(
        _stage_p5, (h4, skip1, x0a, d2_Wr, r(d2_bl), s2_W, r(s2_b)),
        [_row_spec(64), _row_spec(64), _row_spec(48),
         _full_spec((32, 128)), _full_spec((1, 32)),
         _full_spec((32, 32)), _full_spec((1, 32))], (32, 32))
    o = _tc_call(
        _stage_f, (s4, inv, p5, skip2, out_W, r(out_b)),
        [_seg_spec(32), _INV_SPEC, _row_spec(32), _row_spec(32),
         _full_spec((3, 64)), _full_spec((1, 3))], 3)

    return o
